# Initial kernel scaffold; baseline (speedup 1.0000x reference)
import functools

import jax
import jax.numpy as jnp
from jax.experimental import pallas as pl
from jax.experimental.pallas import tpu as pltpu

_B, _C, _H, _W = 4, 96, 224, 224
_L = _H * _W
_E = 8
_TL = 3584
_NL = _L // _TL


def _pool_kernel(x_ref, out_ref):
    j = pl.program_id(1)

    @pl.when(j == 0)
    def _():
        out_ref[...] = jnp.zeros_like(out_ref)

    s = jnp.sum(x_ref[...], axis=2, keepdims=True)
    out_ref[...] += s * (1.0 / _L)


def _moe_kernel(ew_ref, x0_ref, xt_ref, xf_ref, xtf_ref, we_ref, bet_ref,
                ewt_ref, out_ref):
    b = pl.program_id(0)
    x0 = x0_ref[0]
    xt = xt_ref[0]
    xf = jnp.flip(xf_ref[0], axis=1)
    xtf = jnp.flip(xtf_ref[0], axis=1)
    ins = (x0, xt, xf, xtf)
    acc = x0
    for d in range(4):
        m_d = ew_ref[b, d] * we_ref[d] + ew_ref[b, d + 4] * we_ref[d + 4]
        acc = acc + jax.lax.dot_general(
            m_d, ins[d], (((1,), (0,)), ((), ())),
            preferred_element_type=jnp.float32)
    bias = jax.lax.dot_general(
        bet_ref[...], ewt_ref[...], (((1,), (0,)), ((), ())),
        preferred_element_type=jnp.float32)
    out_ref[0] = acc + bias


def _gate(pooled, wg, bg):
    logits = pooled @ wg.T + bg
    w = jax.nn.softmax(logits.astype(jnp.float32), axis=1)
    topw, topi = jax.lax.top_k(w, 2)
    return jnp.zeros_like(w).at[jnp.arange(_B)[:, None], topi].set(topw)


@jax.jit
def kernel(inputs, Wg, bg, We, be):
    x = inputs.reshape(_B, _C, _L)
    xT = jnp.swapaxes(inputs, 2, 3).reshape(_B, _C, _L)

    pooled = pl.pallas_call(
        _pool_kernel,
        grid=(_B, _NL),
        in_specs=[pl.BlockSpec((1, _C, _TL), lambda b, j: (b, 0, j))],
        out_specs=pl.BlockSpec((1, _C, 1), lambda b, j: (b, 0, 0)),
        out_shape=jax.ShapeDtypeStruct((_B, _C, 1), jnp.float32),
        compiler_params=pltpu.CompilerParams(
            dimension_semantics=("arbitrary", "arbitrary")),
    )(x)[:, :, 0]

    ew = _gate(pooled, Wg, bg)
    ewT = jnp.swapaxes(ew, 0, 1)
    beT = jnp.swapaxes(be, 0, 1)

    out = pl.pallas_call(
        _moe_kernel,
        grid=(_B, _NL),
        in_specs=[
            pl.BlockSpec(memory_space=pltpu.SMEM),
            pl.BlockSpec((1, _C, _TL), lambda b, j: (b, 0, j)),
            pl.BlockSpec((1, _C, _TL), lambda b, j: (b, 0, j)),
            pl.BlockSpec((1, _C, _TL), lambda b, j: (b, 0, _NL - 1 - j)),
            pl.BlockSpec((1, _C, _TL), lambda b, j: (b, 0, _NL - 1 - j)),
            pl.BlockSpec((_E, _C, _C), lambda b, j: (0, 0, 0)),
            pl.BlockSpec((_C, _E), lambda b, j: (0, 0)),
            pl.BlockSpec((_E, 1), lambda b, j: (0, b)),
        ],
        out_specs=pl.BlockSpec((1, _C, _TL), lambda b, j: (b, 0, j)),
        out_shape=jax.ShapeDtypeStruct((_B, _C, _L), jnp.float32),
        compiler_params=pltpu.CompilerParams(
            dimension_semantics=("parallel", "parallel")),
    )(ew, x, xT, x, xT, We, beT, ewT)

    return out.reshape(_B, _C, _H, _W)


# speedup vs baseline: 1.3122x; 1.3122x over previous
import functools

import jax
import jax.numpy as jnp
from jax.experimental import pallas as pl
from jax.experimental.pallas import tpu as pltpu

_B, _C, _H, _W = 4, 96, 224, 224
_L = _H * _W
_E = 8
_TL = 3584
_NL = _L // _TL


def _pool_kernel(x_ref, out_ref):
    j = pl.program_id(1)

    @pl.when(j == 0)
    def _():
        out_ref[...] = jnp.zeros_like(out_ref)

    s = jnp.sum(x_ref[...], axis=2, keepdims=True)
    out_ref[...] += s * (1.0 / _L)


def _moe_kernel(ew_ref, x0_ref, xt_ref, xf_ref, xtf_ref, we_ref, bet_ref,
                ewt_ref, out_ref):
    b = pl.program_id(0)
    x0 = x0_ref[0]
    xt = xt_ref[0]
    xf = xf_ref[0]
    xtf = xtf_ref[0]
    ins = (x0, xt, xf, xtf)
    acc = x0
    for d in range(4):
        m_d = ew_ref[b, d] * we_ref[d] + ew_ref[b, d + 4] * we_ref[d + 4]
        acc = acc + jax.lax.dot_general(
            m_d, ins[d], (((1,), (0,)), ((), ())),
            preferred_element_type=jnp.float32)
    bias = jax.lax.dot_general(
        bet_ref[...], ewt_ref[0], (((1,), (0,)), ((), ())),
        preferred_element_type=jnp.float32)
    out_ref[0] = acc + bias


def _gate(pooled, wg, bg):
    logits = pooled @ wg.T + bg
    w = jax.nn.softmax(logits.astype(jnp.float32), axis=1)
    topw, topi = jax.lax.top_k(w, 2)
    return jnp.zeros_like(w).at[jnp.arange(_B)[:, None], topi].set(topw)


@jax.jit
def kernel(inputs, Wg, bg, We, be):
    x = inputs.reshape(_B, _C, _L)
    xT = jnp.swapaxes(inputs, 2, 3).reshape(_B, _C, _L)
    xF = jnp.flip(x, axis=2)
    xTF = jnp.flip(xT, axis=2)

    pooled = pl.pallas_call(
        _pool_kernel,
        grid=(_B, _NL),
        in_specs=[pl.BlockSpec((1, _C, _TL), lambda b, j: (b, 0, j))],
        out_specs=pl.BlockSpec((1, _C, 1), lambda b, j: (b, 0, 0)),
        out_shape=jax.ShapeDtypeStruct((_B, _C, 1), jnp.float32),
        compiler_params=pltpu.CompilerParams(
            dimension_semantics=("arbitrary", "arbitrary")),
    )(x)[:, :, 0]

    ew = _gate(pooled, Wg, bg)
    ewT = ew.reshape(_B, _E, 1)
    beT = jnp.swapaxes(be, 0, 1)

    out = pl.pallas_call(
        _moe_kernel,
        grid=(_B, _NL),
        in_specs=[
            pl.BlockSpec(memory_space=pltpu.SMEM),
            pl.BlockSpec((1, _C, _TL), lambda b, j: (b, 0, j)),
            pl.BlockSpec((1, _C, _TL), lambda b, j: (b, 0, j)),
            pl.BlockSpec((1, _C, _TL), lambda b, j: (b, 0, j)),
            pl.BlockSpec((1, _C, _TL), lambda b, j: (b, 0, j)),
            pl.BlockSpec((_E, _C, _C), lambda b, j: (0, 0, 0)),
            pl.BlockSpec((_C, _E), lambda b, j: (0, 0)),
            pl.BlockSpec((1, _E, 1), lambda b, j: (b, 0, 0)),
        ],
        out_specs=pl.BlockSpec((1, _C, _TL), lambda b, j: (b, 0, j)),
        out_shape=jax.ShapeDtypeStruct((_B, _C, _L), jnp.float32),
        compiler_params=pltpu.CompilerParams(
            dimension_semantics=("parallel", "parallel")),
    )(ew, x, xT, xF, xTF, We, beT, ewT)

    return out.reshape(_B, _C, _H, _W)


# speedup vs baseline: 3.6553x; 2.7857x over previous
import functools

import jax
import jax.numpy as jnp
from jax.experimental import pallas as pl
from jax.experimental.pallas import tpu as pltpu

_B, _C, _H, _W = 4, 96, 224, 224
_L = _H * _W
_E = 8
_TL = 3584
_NL = _L // _TL


def _pool_kernel(x_ref, out_ref):
    j = pl.program_id(1)

    @pl.when(j == 0)
    def _():
        out_ref[...] = jnp.zeros_like(out_ref)

    s = jnp.sum(x_ref[...], axis=2, keepdims=True)
    out_ref[...] += s * (1.0 / _L)


def _dot(a, b):
    return jax.lax.dot_general(a, b, (((1,), (0,)), ((), ())),
                               preferred_element_type=jnp.float32)


def _moe_kernel(ew_ref, x0_ref, xt_ref, x2_ref, x3_ref, we_ref, bet_ref,
                ewt_ref, jrev_ref, out_ref):
    b = pl.program_id(0)
    x0 = x0_ref[0]
    xt = xt_ref[0]
    m = [ew_ref[b, d] * we_ref[d] + ew_ref[b, d + 4] * we_ref[d + 4]
         for d in range(4)]
    bias = _dot(bet_ref[...], ewt_ref[0])
    out_ref[0] = x0 + _dot(m[0], x0) + _dot(m[1], xt) + bias
    x2 = x2_ref[0]
    x3 = x3_ref[0]
    jrev = jrev_ref[...]
    nck = _TL // 128
    for k in range(nck):
        s2 = x2[:, k * 128:(k + 1) * 128]
        s3 = x3[:, k * 128:(k + 1) * 128]
        z = _dot(m[2], s2) + _dot(m[3], s3)
        lo = (nck - 1 - k) * 128
        out_ref[0, :, lo:lo + 128] += _dot(z, jrev)


def _gate(pooled, wg, bg):
    logits = pooled @ wg.T + bg
    w = jax.nn.softmax(logits.astype(jnp.float32), axis=1)
    topw, topi = jax.lax.top_k(w, 2)
    return jnp.zeros_like(w).at[jnp.arange(_B)[:, None], topi].set(topw)


@jax.jit
def kernel(inputs, Wg, bg, We, be):
    x = inputs.reshape(_B, _C, _L)
    xT = jnp.swapaxes(inputs, 2, 3).reshape(_B, _C, _L)
    jrev = jnp.flip(jnp.eye(128, dtype=jnp.float32), 1)

    pooled = pl.pallas_call(
        _pool_kernel,
        grid=(_B, _NL),
        in_specs=[pl.BlockSpec((1, _C, _TL), lambda b, j: (b, 0, j))],
        out_specs=pl.BlockSpec((1, _C, 1), lambda b, j: (b, 0, 0)),
        out_shape=jax.ShapeDtypeStruct((_B, _C, 1), jnp.float32),
        compiler_params=pltpu.CompilerParams(
            dimension_semantics=("arbitrary", "arbitrary")),
    )(x)[:, :, 0]

    ew = _gate(pooled, Wg, bg)
    ewT = ew.reshape(_B, _E, 1)
    beT = jnp.swapaxes(be, 0, 1)

    out = pl.pallas_call(
        _moe_kernel,
        grid=(_B, _NL),
        in_specs=[
            pl.BlockSpec(memory_space=pltpu.SMEM),
            pl.BlockSpec((1, _C, _TL), lambda b, j: (b, 0, j)),
            pl.BlockSpec((1, _C, _TL), lambda b, j: (b, 0, j)),
            pl.BlockSpec((1, _C, _TL), lambda b, j: (b, 0, _NL - 1 - j)),
            pl.BlockSpec((1, _C, _TL), lambda b, j: (b, 0, _NL - 1 - j)),
            pl.BlockSpec((_E, _C, _C), lambda b, j: (0, 0, 0)),
            pl.BlockSpec((_C, _E), lambda b, j: (0, 0)),
            pl.BlockSpec((1, _E, 1), lambda b, j: (b, 0, 0)),
            pl.BlockSpec((128, 128), lambda b, j: (0, 0)),
        ],
        out_specs=pl.BlockSpec((1, _C, _TL), lambda b, j: (b, 0, j)),
        out_shape=jax.ShapeDtypeStruct((_B, _C, _L), jnp.float32),
        compiler_params=pltpu.CompilerParams(
            dimension_semantics=("parallel", "parallel")),
    )(ew, x, xT, x, xT, We, beT, ewT, jrev)

    return out.reshape(_B, _C, _H, _W)


# speedup vs baseline: 4.9153x; 1.3447x over previous
import functools

import jax
import jax.numpy as jnp
from jax.experimental import pallas as pl
from jax.experimental.pallas import tpu as pltpu

_B, _C, _H, _W = 4, 96, 224, 224
_L = _H * _W
_E = 8
_TL = 3584
_NL = _L // _TL


def _prep_kernel(x_ref, xt_ref, pool_ref):
    blk = x_ref[0]
    xt_ref[0] = jnp.swapaxes(blk, 1, 2)
    pool_ref[...] = jnp.sum(blk, axis=(1, 2))[None, :, None] * (1.0 / _L)


def _dot(a, b):
    return jax.lax.dot_general(a, b, (((1,), (0,)), ((), ())),
                               preferred_element_type=jnp.float32)


def _moe_kernel(ew_ref, x0_ref, xt_ref, x2_ref, x3_ref, we_ref, bet_ref,
                ewt_ref, jrev_ref, out_ref):
    b = pl.program_id(0)
    x0 = x0_ref[0]
    xt = xt_ref[0]
    m = [ew_ref[b, d] * we_ref[d] + ew_ref[b, d + 4] * we_ref[d + 4]
         for d in range(4)]
    bias = _dot(bet_ref[...], ewt_ref[0])
    out_ref[0] = x0 + _dot(m[0], x0) + _dot(m[1], xt) + bias
    x2 = x2_ref[0]
    x3 = x3_ref[0]
    jrev = jrev_ref[...]
    nck = _TL // 128
    for k in range(nck):
        s2 = x2[:, k * 128:(k + 1) * 128]
        s3 = x3[:, k * 128:(k + 1) * 128]
        z = _dot(m[2], s2) + _dot(m[3], s3)
        lo = (nck - 1 - k) * 128
        out_ref[0, :, lo:lo + 128] += _dot(z, jrev)


def _gate(pooled, wg, bg):
    logits = pooled @ wg.T + bg
    w = jax.nn.softmax(logits.astype(jnp.float32), axis=1)
    topw, topi = jax.lax.top_k(w, 2)
    return jnp.zeros_like(w).at[jnp.arange(_B)[:, None], topi].set(topw)


@jax.jit
def kernel(inputs, Wg, bg, We, be):
    x = inputs.reshape(_B, _C, _L)
    jrev = jnp.flip(jnp.eye(128, dtype=jnp.float32), 1)

    _CB = 32
    xT4, pooled = pl.pallas_call(
        _prep_kernel,
        grid=(_B, _C // _CB),
        in_specs=[pl.BlockSpec((1, _CB, _H, _W), lambda b, c: (b, c, 0, 0))],
        out_specs=[
            pl.BlockSpec((1, _CB, _W, _H), lambda b, c: (b, c, 0, 0)),
            pl.BlockSpec((1, _CB, 1), lambda b, c: (b, c, 0)),
        ],
        out_shape=[
            jax.ShapeDtypeStruct((_B, _C, _W, _H), jnp.float32),
            jax.ShapeDtypeStruct((_B, _C, 1), jnp.float32),
        ],
        compiler_params=pltpu.CompilerParams(
            dimension_semantics=("arbitrary", "arbitrary")),
    )(inputs)
    xT = xT4.reshape(_B, _C, _L)
    pooled = pooled[:, :, 0]

    ew = _gate(pooled, Wg, bg)
    ewT = ew.reshape(_B, _E, 1)
    beT = jnp.swapaxes(be, 0, 1)

    out = pl.pallas_call(
        _moe_kernel,
        grid=(_B, _NL),
        in_specs=[
            pl.BlockSpec(memory_space=pltpu.SMEM),
            pl.BlockSpec((1, _C, _TL), lambda b, j: (b, 0, j)),
            pl.BlockSpec((1, _C, _TL), lambda b, j: (b, 0, j)),
            pl.BlockSpec((1, _C, _TL), lambda b, j: (b, 0, _NL - 1 - j)),
            pl.BlockSpec((1, _C, _TL), lambda b, j: (b, 0, _NL - 1 - j)),
            pl.BlockSpec((_E, _C, _C), lambda b, j: (0, 0, 0)),
            pl.BlockSpec((_C, _E), lambda b, j: (0, 0)),
            pl.BlockSpec((1, _E, 1), lambda b, j: (b, 0, 0)),
            pl.BlockSpec((128, 128), lambda b, j: (0, 0)),
        ],
        out_specs=pl.BlockSpec((1, _C, _TL), lambda b, j: (b, 0, j)),
        out_shape=jax.ShapeDtypeStruct((_B, _C, _L), jnp.float32),
        compiler_params=pltpu.CompilerParams(
            dimension_semantics=("parallel", "parallel")),
    )(ew, x, xT, x, xT, We, beT, ewT, jrev)

    return out.reshape(_B, _C, _H, _W)


# speedup vs baseline: 6.1593x; 1.2531x over previous
import functools

import jax
import jax.numpy as jnp
from jax.experimental import pallas as pl
from jax.experimental.pallas import tpu as pltpu

_B, _C, _H, _W = 4, 96, 224, 224
_L = _H * _W
_E = 8
_TL = 3584
_NL = _L // _TL


def _prep_kernel(x_ref, xt_ref, pool_ref):
    blk = x_ref[0]
    xt_ref[0] = jnp.swapaxes(blk, 1, 2)
    pool_ref[...] = jnp.sum(blk, axis=(1, 2))[None, :, None] * (1.0 / _L)


def _dot(a, b):
    return jax.lax.dot_general(a, b, (((1,), (0,)), ((), ())),
                               preferred_element_type=jnp.float32)


def _moe_kernel(ew_ref, x0_ref, xt_ref, x2_ref, x3_ref, we_ref, bet_ref,
                ewt_ref, jrev_ref, out_ref):
    b = pl.program_id(0)
    s = pl.program_id(2)
    m = [ew_ref[b, d] * we_ref[d] + ew_ref[b, d + 4] * we_ref[d + 4]
         for d in range(4)]
    bias = _dot(bet_ref[...], ewt_ref[0])
    jrev = jrev_ref[...]
    nck = _TL // 128

    def emit(a, bt, c, d):
        out_ref[0] = a + _dot(m[0], a) + _dot(m[1], bt) + bias
        z = _dot(m[2], c) + _dot(m[3], d)
        for k in range(nck):
            lo = (nck - 1 - k) * 128
            out_ref[0, :, lo:lo + 128] += _dot(z[:, k * 128:(k + 1) * 128],
                                               jrev)

    @pl.when(s == 0)
    def _():
        emit(x0_ref[0], xt_ref[0], x2_ref[0], x3_ref[0])

    @pl.when(s == 1)
    def _():
        emit(x2_ref[0], x3_ref[0], x0_ref[0], xt_ref[0])


def _gate(pooled, wg, bg):
    logits = pooled @ wg.T + bg
    w = jax.nn.softmax(logits.astype(jnp.float32), axis=1)
    topw, topi = jax.lax.top_k(w, 2)
    return jnp.zeros_like(w).at[jnp.arange(_B)[:, None], topi].set(topw)


@jax.jit
def kernel(inputs, Wg, bg, We, be):
    x = inputs.reshape(_B, _C, _L)
    jrev = jnp.flip(jnp.eye(128, dtype=jnp.float32), 1)

    _CB = 32
    xT4, pooled = pl.pallas_call(
        _prep_kernel,
        grid=(_B, _C // _CB),
        in_specs=[pl.BlockSpec((1, _CB, _H, _W), lambda b, c: (b, c, 0, 0))],
        out_specs=[
            pl.BlockSpec((1, _CB, _W, _H), lambda b, c: (b, c, 0, 0)),
            pl.BlockSpec((1, _CB, 1), lambda b, c: (b, c, 0)),
        ],
        out_shape=[
            jax.ShapeDtypeStruct((_B, _C, _W, _H), jnp.float32),
            jax.ShapeDtypeStruct((_B, _C, 1), jnp.float32),
        ],
        compiler_params=pltpu.CompilerParams(
            dimension_semantics=("arbitrary", "arbitrary")),
    )(inputs)
    xT = xT4.reshape(_B, _C, _L)
    pooled = pooled[:, :, 0]

    ew = _gate(pooled, Wg, bg)
    ewT = ew.reshape(_B, _E, 1)
    beT = jnp.swapaxes(be, 0, 1)

    out = pl.pallas_call(
        _moe_kernel,
        grid=(_B, _NL // 2, 2),
        in_specs=[
            pl.BlockSpec(memory_space=pltpu.SMEM),
            pl.BlockSpec((1, _C, _TL), lambda b, j, s: (b, 0, j)),
            pl.BlockSpec((1, _C, _TL), lambda b, j, s: (b, 0, j)),
            pl.BlockSpec((1, _C, _TL),
                         lambda b, j, s: (b, 0, _NL - 1 - j)),
            pl.BlockSpec((1, _C, _TL),
                         lambda b, j, s: (b, 0, _NL - 1 - j)),
            pl.BlockSpec((_E, _C, _C), lambda b, j, s: (0, 0, 0)),
            pl.BlockSpec((_C, _E), lambda b, j, s: (0, 0)),
            pl.BlockSpec((1, _E, 1), lambda b, j, s: (b, 0, 0)),
            pl.BlockSpec((128, 128), lambda b, j, s: (0, 0)),
        ],
        out_specs=pl.BlockSpec(
            (1, _C, _TL),
            lambda b, j, s: (b, 0, j + s * (_NL - 1 - 2 * j))),
        out_shape=jax.ShapeDtypeStruct((_B, _C, _L), jnp.float32),
        compiler_params=pltpu.CompilerParams(
            dimension_semantics=("parallel", "parallel", "arbitrary")),
    )(ew, x, xT, x, xT, We, beT, ewT, jrev)

    return out.reshape(_B, _C, _H, _W)
